# baseline (speedup 1.0000x reference)
import functools

import jax
import jax.numpy as jnp
from jax import lax
from jax.experimental import pallas as pl
from jax.experimental.pallas import tpu as pltpu
from jax.experimental.pallas import tpu_sc as plsc

N = 10000
D = 128
H = D // 2
E = 320000
T = 475
TP = 480
NS, L = 16, 16
CHUNK = 128
NCHUNKS = E // CHUNK
KPT = (NCHUNKS + NS - 1) // NS
RH = N // 2
NPH = 5376
RPT = NPH // NS
TRASH = RH


def _proj_body(x_ref, vp_ref, vg_ref, bm_ref, bg_ref, out_ref):
    x = x_ref[...]
    xp = jnp.dot(x, vp_ref[...], preferred_element_type=jnp.float32)
    xg = jnp.dot(x, vg_ref[...], preferred_element_type=jnp.float32)
    xp = jnp.maximum(xp + bm_ref[...], 0.0)
    xg = xg + bg_ref[...]
    for h in range(2):
        out_ref[h, :, :H] = xp[:, h * H:(h + 1) * H]
        out_ref[h, :, H:] = xg[:, h * H:(h + 1) * H]


def _project(x, vp, vg, bm, bg):
    return pl.pallas_call(
        _proj_body,
        out_shape=jax.ShapeDtypeStruct((2, N, D), jnp.float32),
    )(x, vp, vg, bm.reshape(1, D), bg.reshape(1, D))


def _make_edge_body(h, p):
    soff = h * N
    roff = p * RH

    def _edge_body(cmb_hbm, egt_hbm, sidx_hbm, ridx_hbm, tidx_hbm, zer_hbm,
                   acc_out,
                   sidx_v, ridx_v, tidx_v, buf_v, egt_v, acc_sh, sem0):
        s = lax.axis_index("s")

        pltpu.sync_copy(egt_hbm, egt_v)
        rows = pl.ds(s * RPT, RPT)
        pltpu.sync_copy(zer_hbm.at[rows], acc_sh.at[rows])
        plsc.subcore_barrier()

        def chunk(k, carry):
            cid = s + NS * k

            @pl.when(cid < NCHUNKS)
            def _():
                base = cid * CHUNK
                pltpu.sync_copy(sidx_hbm.at[pl.ds(base, CHUNK)], sidx_v)
                pltpu.sync_copy(tidx_hbm.at[pl.ds(base, CHUNK)], tidx_v)
                pltpu.sync_copy(ridx_hbm.at[pl.ds(base, CHUNK)], ridx_v)
                for l in range(CHUNK // L):
                    sl = pl.ds(l * L, L)
                    if soff:
                        sidx_v[sl] = sidx_v[sl] + soff
                    r16 = ridx_v[sl]
                    if p == 0:
                        ridx_v[sl] = jnp.where(r16 >= RH, TRASH, r16)
                    else:
                        rl = r16 - roff
                        ridx_v[sl] = jnp.where(rl < 0, TRASH, rl)
                pltpu.async_copy(cmb_hbm.at[sidx_v], buf_v, sem0).wait()

                def blk(b, carry2):
                    tvec = tidx_v[pl.ds(b * L, L)]
                    for i in range(L):
                        j = b * L + i
                        t = tvec[i]
                        for l in range(H // L):
                            slp = pl.ds(l * L, L)
                            slg = pl.ds(H + l * L, L)
                            e = jnp.exp(buf_v[j, slg] + egt_v[t, slp])
                            buf_v[j, slg] = e
                            buf_v[j, slp] = buf_v[j, slp] * e
                    return carry2

                lax.fori_loop(0, CHUNK // L, blk, 0)
                pltpu.sync_copy(buf_v, acc_sh.at[ridx_v], add=True)

            return carry

        lax.fori_loop(0, KPT, chunk, 0)
        plsc.subcore_barrier()
        pltpu.sync_copy(acc_sh.at[rows], acc_out.at[rows])

    return _edge_body


def _make_edge_kernel(h, p):
    return functools.partial(
        pl.kernel,
        out_type=jax.ShapeDtypeStruct((NPH, D), jnp.float32),
        mesh=plsc.VectorSubcoreMesh(
            core_axis_name="c", subcore_axis_name="s", num_cores=1,
            num_subcores=NS),
        scratch_types=[
            pltpu.VMEM((CHUNK,), jnp.int32),
            pltpu.VMEM((CHUNK,), jnp.int32),
            pltpu.VMEM((CHUNK,), jnp.int32),
            pltpu.VMEM((CHUNK, D), jnp.float32),
            pltpu.VMEM((TP, H), jnp.float32),
            pltpu.VMEM_SHARED((NPH, D), jnp.float32),
            pltpu.SemaphoreType.DMA,
        ],
    )(_make_edge_body(h, p))


_edge_kernels = {(h, p): _make_edge_kernel(h, p)
                 for h in range(2) for p in range(2)}


def _div_body(a00_ref, a01_ref, a10_ref, a11_ref, out_ref):
    refs = {(0, 0): a00_ref, (0, 1): a01_ref,
            (1, 0): a10_ref, (1, 1): a11_ref}
    for h in range(2):
        for p in range(2):
            ref = refs[(h, p)]
            n = ref[:RH, :H]
            d = ref[:RH, H:]
            safe = jnp.where(d > 0.0, d, 1.0)
            out_ref[p * RH:(p + 1) * RH, h * H:(h + 1) * H] = jnp.where(
                d > 0.0, n / safe, 0.0)


def _divide(a00, a01, a10, a11):
    return pl.pallas_call(
        _div_body,
        out_shape=jax.ShapeDtypeStruct((N, D), jnp.float32),
    )(a00, a01, a10, a11)


def kernel(x, sender_idx, receiver_idx, type_idx, V_proj_sender,
           V_gate_sender, E_gate, B_message, B_gate_pre):
    cmb = _project(x, V_proj_sender, V_gate_sender, B_message,
                   B_gate_pre).reshape(2 * N, D)
    egs = E_gate.reshape(T, 2, H).transpose(1, 0, 2)
    egt = jnp.zeros((2, TP, H), jnp.float32).at[:, :T, :].set(egs)
    zer = jnp.zeros((NPH, D), jnp.float32)
    idx = (sender_idx, receiver_idx, type_idx)
    acc = {(h, p): _edge_kernels[(h, p)](cmb, egt[h], *idx, zer)
           for h in range(2) for p in range(2)}
    return _divide(acc[(0, 0)], acc[(0, 1)], acc[(1, 0)], acc[(1, 1)])


# speedup vs baseline: 1.2058x; 1.2058x over previous
import functools

import jax
import jax.numpy as jnp
from jax import lax
from jax.experimental import pallas as pl
from jax.experimental.pallas import tpu as pltpu
from jax.experimental.pallas import tpu_sc as plsc

N = 10000
D = 128
H = D // 2
E = 320000
T = 475
TP = 480
NS, L = 16, 16
CHUNK = 96
CPT = 216
EP = NS * CPT * CHUNK
NCH = NS * CPT
RH = N // 2
NPH = 5120
RPT = NPH // NS
TRASH = RH


def _proj_body(x_ref, vp_ref, vg_ref, bm_ref, bg_ref, out_ref):
    x = x_ref[...]
    xp = jnp.dot(x, vp_ref[...], preferred_element_type=jnp.float32)
    xg = jnp.dot(x, vg_ref[...], preferred_element_type=jnp.float32)
    xp = jnp.maximum(xp + bm_ref[...], 0.0)
    xg = xg + bg_ref[...]
    for h in range(2):
        out_ref[h, :, :H] = xp[:, h * H:(h + 1) * H]
        out_ref[h, :, H:] = xg[:, h * H:(h + 1) * H]


def _project(x, vp, vg, bm, bg):
    return pl.pallas_call(
        _proj_body,
        out_shape=jax.ShapeDtypeStruct((2, N, D), jnp.float32),
    )(x, vp, vg, bm.reshape(1, D), bg.reshape(1, D))


def _edge_body(cmb_hbm, egt_hbm, idx_hbm, zer_hbm,
               acc_out,
               idx0_v, idx1_v, egt_v, buf0_v, buf1_v, sadj0_v, sadj1_v,
               radj0_v, radj1_v, acc_sh, semg0, semg1, semi0, semi1):
    s = lax.axis_index("s")
    arows = pl.ds(s * RPT, RPT)
    cbase = s * CPT

    bufs = (buf0_v, buf1_v)
    idxs = (idx0_v, idx1_v)
    sadjs = (sadj0_v, sadj1_v)
    radjs = (radj0_v, radj1_v)
    semg = (semg0, semg1)
    semi = (semi0, semi1)

    def fire_idx(k, slot):
        pltpu.async_copy(idx_hbm.at[cbase + k], idxs[slot], semi[slot])

    def wait_idx(slot, k):
        pltpu.make_async_copy(
            idx_hbm.at[cbase + k], idxs[slot], semi[slot]).wait()

    def fire_gather(soff, slot):
        sadj = sadjs[slot]
        for l in range(CHUNK // L):
            sl = pl.ds(l * L, L)
            sadj[sl] = idxs[slot][0, sl] + soff
        pltpu.async_copy(cmb_hbm.at[sadj], bufs[slot], semg[slot])

    def wait_gather(slot):
        pltpu.make_async_copy(
            cmb_hbm.at[sadjs[slot]], bufs[slot], semg[slot]).wait()

    def phase(ph, carry):
        h = ph % 2
        p = ph // 2
        soff = h * N
        roff = p * RH

        pltpu.sync_copy(egt_hbm.at[h], egt_v)
        pltpu.sync_copy(zer_hbm.at[arows], acc_sh.at[arows])
        plsc.subcore_barrier()

        pltpu.sync_copy(idx_hbm.at[cbase], idx0_v)
        fire_gather(soff, 0)
        fire_idx(1, 1)

        def do_chunk(k, si):
            so = 1 - si

            @pl.when(k + 1 < CPT)
            def _():
                wait_idx(so, k + 1)
                fire_gather(soff, so)
            wait_gather(si)

            buf_v = bufs[si]
            idx_v = idxs[si]
            radj_v = radjs[si]
            for l in range(CHUNK // L):
                sl = pl.ds(l * L, L)
                rl = idx_v[1, sl] - roff
                oob = (rl < 0) | (rl >= RH)
                radj_v[0, sl] = jnp.where(oob, TRASH, rl)

            def blk(b, carry3):
                tvec = idx_v[2, pl.ds(b * L, L)]
                for i in range(L):
                    j = b * L + i
                    t = tvec[i]
                    for l in range(H // L):
                        slp = pl.ds(l * L, L)
                        slg = pl.ds(H + l * L, L)
                        e = jnp.exp(buf_v[j, slg] + egt_v[t, slp])
                        buf_v[j, slg] = e
                        buf_v[j, slp] = buf_v[j, slp] * e
                return carry3

            lax.fori_loop(0, CHUNK // L, blk, 0)
            pltpu.sync_copy(buf_v, acc_sh.at[radj_v.at[0]], add=True)

            @pl.when(k + 2 < CPT)
            def _i():
                fire_idx(k + 2, si)

        def chunk2(m, carry2):
            do_chunk(2 * m, 0)
            do_chunk(2 * m + 1, 1)
            return carry2

        lax.fori_loop(0, CPT // 2, chunk2, 0)
        plsc.subcore_barrier()
        pltpu.sync_copy(acc_sh.at[arows], acc_out.at[ph, arows])
        plsc.subcore_barrier()
        return carry

    lax.fori_loop(0, 4, phase, 0)


_edge_kernel = functools.partial(
    pl.kernel,
    out_type=jax.ShapeDtypeStruct((4, NPH, D), jnp.float32),
    mesh=plsc.VectorSubcoreMesh(
        core_axis_name="c", subcore_axis_name="s", num_cores=1,
        num_subcores=NS),
    scratch_types=[
        pltpu.VMEM((3, CHUNK), jnp.int32),
        pltpu.VMEM((3, CHUNK), jnp.int32),
        pltpu.VMEM((TP, H), jnp.float32),
        pltpu.VMEM((CHUNK, D), jnp.float32),
        pltpu.VMEM((CHUNK, D), jnp.float32),
        pltpu.VMEM((CHUNK,), jnp.int32),
        pltpu.VMEM((CHUNK,), jnp.int32),
        pltpu.VMEM((1, CHUNK), jnp.int32),
        pltpu.VMEM((1, CHUNK), jnp.int32),
        pltpu.VMEM_SHARED((NPH, D), jnp.float32),
        pltpu.SemaphoreType.DMA,
        pltpu.SemaphoreType.DMA,
        pltpu.SemaphoreType.DMA,
        pltpu.SemaphoreType.DMA,
    ],
)(_edge_body)


def _div_body(acc_ref, out_ref):
    for ph in range(4):
        h, p = ph % 2, ph // 2
        n = acc_ref[ph, :RH, :H]
        d = acc_ref[ph, :RH, H:]
        safe = jnp.where(d > 0.0, d, 1.0)
        out_ref[p * RH:(p + 1) * RH, h * H:(h + 1) * H] = jnp.where(
            d > 0.0, n / safe, 0.0)


def _divide(acc):
    return pl.pallas_call(
        _div_body,
        out_shape=jax.ShapeDtypeStruct((N, D), jnp.float32),
    )(acc)


def kernel(x, sender_idx, receiver_idx, type_idx, V_proj_sender,
           V_gate_sender, E_gate, B_message, B_gate_pre):
    cmb = _project(x, V_proj_sender, V_gate_sender, B_message,
                   B_gate_pre).reshape(2 * N, D)
    egs = E_gate.reshape(T, 2, H).transpose(1, 0, 2)
    egt = jnp.zeros((2, TP, H), jnp.float32).at[:, :T, :].set(egs)
    zer = jnp.zeros((NPH, D), jnp.float32)
    pad = EP - E
    sidx = jnp.concatenate(
        [sender_idx, jnp.zeros((pad,), jnp.int32)]).reshape(NCH, CHUNK)
    ridx = jnp.concatenate(
        [receiver_idx, jnp.full((pad,), N, jnp.int32)]).reshape(NCH, CHUNK)
    tidx = jnp.concatenate(
        [type_idx, jnp.zeros((pad,), jnp.int32)]).reshape(NCH, CHUNK)
    idx = jnp.stack([sidx, ridx, tidx], axis=1)
    acc = _edge_kernel(cmb, egt, idx, zer)
    return _divide(acc)


# speedup vs baseline: 2.3738x; 1.9687x over previous
import functools

import jax
import jax.numpy as jnp
from jax import lax
from jax.experimental import pallas as pl
from jax.experimental.pallas import tpu as pltpu
from jax.experimental.pallas import tpu_sc as plsc

N = 10000
D = 128
H = D // 2
E = 320000
T = 475
TP = 480
NS, L = 16, 16
CHUNK = 96
CPT = 216
EP = NS * CPT * CHUNK
NCH = NS * CPT
RH = N // 2
NPH = 5120
RPT = NPH // NS
TRASH = RH


def _proj_body(x_ref, vp_ref, vg_ref, bm_ref, bg_ref, out_ref):
    x = x_ref[...]
    xp = jnp.dot(x, vp_ref[...], preferred_element_type=jnp.float32)
    xg = jnp.dot(x, vg_ref[...], preferred_element_type=jnp.float32)
    xp = jnp.maximum(xp + bm_ref[...], 0.0)
    xg = xg + bg_ref[...]
    for h in range(2):
        out_ref[h, :, :H] = xp[:, h * H:(h + 1) * H]
        out_ref[h, :, H:] = xg[:, h * H:(h + 1) * H]


def _project(x, vp, vg, bm, bg):
    return pl.pallas_call(
        _proj_body,
        out_shape=jax.ShapeDtypeStruct((2, N, D), jnp.float32),
    )(x, vp, vg, bm.reshape(1, D), bg.reshape(1, D))


def _edge_body(cmb_hbm, egt_hbm, idx_hbm, zer_hbm,
               acc_out,
               idx0_v, idx1_v, egt_v, buf0_v, buf1_v, sadj0_v, sadj1_v,
               radj0_v, radj1_v, acc_sh, semg0, semg1, semi0, semi1):
    s = lax.axis_index("s")
    arows = pl.ds(s * RPT, RPT)
    cbase = s * CPT

    bufs = (buf0_v, buf1_v)
    idxs = (idx0_v, idx1_v)
    sadjs = (sadj0_v, sadj1_v)
    radjs = (radj0_v, radj1_v)
    semg = (semg0, semg1)
    semi = (semi0, semi1)

    def fire_idx(k, slot):
        pltpu.async_copy(idx_hbm.at[cbase + k], idxs[slot], semi[slot])

    def wait_idx(slot, k):
        pltpu.make_async_copy(
            idx_hbm.at[cbase + k], idxs[slot], semi[slot]).wait()

    def fire_gather(soff, slot):
        sadj = sadjs[slot]
        for l in range(CHUNK // L):
            sl = pl.ds(l * L, L)
            sadj[sl] = idxs[slot][0, sl] + soff
        pltpu.async_copy(cmb_hbm.at[sadj], bufs[slot], semg[slot])

    def wait_gather(slot):
        pltpu.make_async_copy(
            cmb_hbm.at[sadjs[slot]], bufs[slot], semg[slot]).wait()

    c = lax.axis_index("c")

    def phase(p, carry):
        soff = c * N
        roff = p * RH

        pltpu.sync_copy(egt_hbm.at[c], egt_v)
        pltpu.sync_copy(zer_hbm.at[arows], acc_sh.at[arows])
        plsc.subcore_barrier()

        pltpu.sync_copy(idx_hbm.at[cbase], idx0_v)
        fire_gather(soff, 0)
        fire_idx(1, 1)

        def do_chunk(k, si):
            so = 1 - si

            @pl.when(k + 1 < CPT)
            def _():
                wait_idx(so, k + 1)
                fire_gather(soff, so)
            wait_gather(si)

            buf_v = bufs[si]
            idx_v = idxs[si]
            radj_v = radjs[si]
            for l in range(CHUNK // L):
                sl = pl.ds(l * L, L)
                rl = idx_v[1, sl] - roff
                oob = (rl < 0) | (rl >= RH)
                radj_v[0, sl] = jnp.where(oob, TRASH, rl)

            def blk(b, carry3):
                tvec = idx_v[2, pl.ds(b * L, L)]
                for i in range(L):
                    j = b * L + i
                    t = tvec[i]
                    for l in range(H // L):
                        slp = pl.ds(l * L, L)
                        slg = pl.ds(H + l * L, L)
                        e = jnp.exp(buf_v[j, slg] + egt_v[t, slp])
                        buf_v[j, slg] = e
                        buf_v[j, slp] = buf_v[j, slp] * e
                return carry3

            lax.fori_loop(0, CHUNK // L, blk, 0)
            pltpu.sync_copy(buf_v, acc_sh.at[radj_v.at[0]], add=True)

            @pl.when(k + 2 < CPT)
            def _i():
                fire_idx(k + 2, si)

        def chunk2(m, carry2):
            do_chunk(2 * m, 0)
            do_chunk(2 * m + 1, 1)
            return carry2

        lax.fori_loop(0, CPT // 2, chunk2, 0)
        plsc.subcore_barrier()
        pltpu.sync_copy(acc_sh.at[arows], acc_out.at[c, p, arows])
        plsc.subcore_barrier()
        return carry

    lax.fori_loop(0, 2, phase, 0)


_edge_kernel = functools.partial(
    pl.kernel,
    out_type=jax.ShapeDtypeStruct((2, 2, NPH, D), jnp.float32),
    mesh=plsc.VectorSubcoreMesh(
        core_axis_name="c", subcore_axis_name="s", num_cores=2,
        num_subcores=NS),
    scratch_types=[
        pltpu.VMEM((3, CHUNK), jnp.int32),
        pltpu.VMEM((3, CHUNK), jnp.int32),
        pltpu.VMEM((TP, H), jnp.float32),
        pltpu.VMEM((CHUNK, D), jnp.float32),
        pltpu.VMEM((CHUNK, D), jnp.float32),
        pltpu.VMEM((CHUNK,), jnp.int32),
        pltpu.VMEM((CHUNK,), jnp.int32),
        pltpu.VMEM((1, CHUNK), jnp.int32),
        pltpu.VMEM((1, CHUNK), jnp.int32),
        pltpu.VMEM_SHARED((NPH, D), jnp.float32),
        pltpu.SemaphoreType.DMA,
        pltpu.SemaphoreType.DMA,
        pltpu.SemaphoreType.DMA,
        pltpu.SemaphoreType.DMA,
    ],
)(_edge_body)


def _div_body(acc_ref, out_ref):
    for h in range(2):
      for p in range(2):
        n = acc_ref[h, p, :RH, :H]
        d = acc_ref[h, p, :RH, H:]
        safe = jnp.where(d > 0.0, d, 1.0)
        out_ref[p * RH:(p + 1) * RH, h * H:(h + 1) * H] = jnp.where(
            d > 0.0, n / safe, 0.0)


def _divide(acc):
    return pl.pallas_call(
        _div_body,
        out_shape=jax.ShapeDtypeStruct((N, D), jnp.float32),
    )(acc)


def kernel(x, sender_idx, receiver_idx, type_idx, V_proj_sender,
           V_gate_sender, E_gate, B_message, B_gate_pre):
    cmb = _project(x, V_proj_sender, V_gate_sender, B_message,
                   B_gate_pre).reshape(2 * N, D)
    egs = E_gate.reshape(T, 2, H).transpose(1, 0, 2)
    egt = jnp.zeros((2, TP, H), jnp.float32).at[:, :T, :].set(egs)
    zer = jnp.zeros((NPH, D), jnp.float32)
    pad = EP - E
    sidx = jnp.concatenate(
        [sender_idx, jnp.zeros((pad,), jnp.int32)]).reshape(NCH, CHUNK)
    ridx = jnp.concatenate(
        [receiver_idx, jnp.full((pad,), N, jnp.int32)]).reshape(NCH, CHUNK)
    tidx = jnp.concatenate(
        [type_idx, jnp.zeros((pad,), jnp.int32)]).reshape(NCH, CHUNK)
    idx = jnp.stack([sidx, ridx, tidx], axis=1)
    acc = _edge_kernel(cmb, egt, idx, zer)
    return _divide(acc)


# speedup vs baseline: 3.9506x; 1.6643x over previous
import functools

import jax
import jax.numpy as jnp
from jax import lax
from jax.experimental import pallas as pl
from jax.experimental.pallas import tpu as pltpu
from jax.experimental.pallas import tpu_sc as plsc

N = 10000
D = 128
H = D // 2
E = 320000
T = 475
TP = 480
NS, L = 16, 16
CHUNK = 96
CPT = 216
EP = NS * CPT * CHUNK
NCH = NS * CPT
RH = N // 2
NPH = 5120
RPT = NPH // NS
TRASH = RH


def _proj_body(x_ref, vp_ref, vg_ref, bm_ref, bg_ref, out_ref):
    x = x_ref[...]
    xp = jnp.dot(x, vp_ref[...], preferred_element_type=jnp.float32)
    xg = jnp.dot(x, vg_ref[...], preferred_element_type=jnp.float32)
    xp = jnp.maximum(xp + bm_ref[...], 0.0)
    xg = xg + bg_ref[...]
    for h in range(2):
        out_ref[h, :, :H] = xp[:, h * H:(h + 1) * H]
        out_ref[h, :, H:] = xg[:, h * H:(h + 1) * H]


def _project(x, vp, vg, bm, bg):
    return pl.pallas_call(
        _proj_body,
        out_shape=jax.ShapeDtypeStruct((2, N, D), jnp.float32),
    )(x, vp, vg, bm.reshape(1, D), bg.reshape(1, D))


def _edge_body(cmb_hbm, egt_hbm, idx_hbm, zer_hbm,
               acc_out,
               idx0_v, idx1_v, egt_v, buf0_v, buf1_v, sadj0_v, sadj1_v,
               radj0_v, radj1_v, acc_sh, semg0, semg1, semi0, semi1):
    s = lax.axis_index("s")
    arows = pl.ds(s * RPT, RPT)
    cbase = s * CPT

    bufs = (buf0_v, buf1_v)
    idxs = (idx0_v, idx1_v)
    sadjs = (sadj0_v, sadj1_v)
    radjs = (radj0_v, radj1_v)
    semg = (semg0, semg1)
    semi = (semi0, semi1)

    def fire_idx(k, slot):
        pltpu.async_copy(idx_hbm.at[cbase + k], idxs[slot], semi[slot])

    def wait_idx(slot, k):
        pltpu.make_async_copy(
            idx_hbm.at[cbase + k], idxs[slot], semi[slot]).wait()

    def fire_gather(soff, slot):
        sadj = sadjs[slot]
        for l in range(CHUNK // L):
            sl = pl.ds(l * L, L)
            sadj[sl] = idxs[slot][0, sl] + soff
        pltpu.async_copy(cmb_hbm.at[sadj], bufs[slot], semg[slot])

    def wait_gather(slot):
        pltpu.make_async_copy(
            cmb_hbm.at[sadjs[slot]], bufs[slot], semg[slot]).wait()

    c = lax.axis_index("c")

    def phase(p, carry):
        soff = c * N
        roff = p * RH

        pltpu.sync_copy(egt_hbm.at[c], egt_v)
        pltpu.sync_copy(zer_hbm.at[arows], acc_sh.at[arows])
        plsc.subcore_barrier()

        pltpu.sync_copy(idx_hbm.at[cbase], idx0_v)
        fire_gather(soff, 0)
        fire_idx(1, 1)

        def do_chunk(k, si):
            so = 1 - si

            @pl.when(k + 1 < CPT)
            def _():
                wait_idx(so, k + 1)
                fire_gather(soff, so)
            wait_gather(si)

            buf_v = bufs[si]
            idx_v = idxs[si]
            radj_v = radjs[si]
            for l in range(CHUNK // L):
                sl = pl.ds(l * L, L)
                rl = idx_v[1, sl] - roff
                oob = (rl < 0) | (rl >= RH)
                radj_v[0, sl] = jnp.where(oob, TRASH, rl)

            def blk(b, carry3):
                tvec = idx_v[2, pl.ds(b * L, L)]
                nsl = H // L
                for i0 in range(0, L, 2):
                    rows = []
                    for i in (i0, i0 + 1):
                        j = b * L + i
                        t = tvec[i]
                        g = [buf_v[j, pl.ds(H + l * L, L)] for l in range(nsl)]
                        eg = [egt_v[t, pl.ds(l * L, L)] for l in range(nsl)]
                        m = [buf_v[j, pl.ds(l * L, L)] for l in range(nsl)]
                        rows.append((j, g, eg, m))
                    ev = [[jnp.exp(g[l] + eg[l]) for l in range(nsl)]
                          for (j, g, eg, m) in rows]
                    for (j, g, eg, m), e in zip(rows, ev):
                        for l in range(nsl):
                            buf_v[j, pl.ds(H + l * L, L)] = e[l]
                            buf_v[j, pl.ds(l * L, L)] = m[l] * e[l]
                return carry3

            lax.fori_loop(0, CHUNK // L, blk, 0)
            pltpu.sync_copy(buf_v, acc_sh.at[radj_v.at[0]], add=True)

            @pl.when(k + 2 < CPT)
            def _i():
                fire_idx(k + 2, si)

        def chunk2(m, carry2):
            do_chunk(2 * m, 0)
            do_chunk(2 * m + 1, 1)
            return carry2

        lax.fori_loop(0, CPT // 2, chunk2, 0)
        plsc.subcore_barrier()
        pltpu.sync_copy(acc_sh.at[arows], acc_out.at[c, p, arows])
        plsc.subcore_barrier()
        return carry

    lax.fori_loop(0, 2, phase, 0)


_edge_kernel = functools.partial(
    pl.kernel,
    out_type=jax.ShapeDtypeStruct((2, 2, NPH, D), jnp.float32),
    mesh=plsc.VectorSubcoreMesh(
        core_axis_name="c", subcore_axis_name="s", num_cores=2,
        num_subcores=NS),
    scratch_types=[
        pltpu.VMEM((3, CHUNK), jnp.int32),
        pltpu.VMEM((3, CHUNK), jnp.int32),
        pltpu.VMEM((TP, H), jnp.float32),
        pltpu.VMEM((CHUNK, D), jnp.float32),
        pltpu.VMEM((CHUNK, D), jnp.float32),
        pltpu.VMEM((CHUNK,), jnp.int32),
        pltpu.VMEM((CHUNK,), jnp.int32),
        pltpu.VMEM((1, CHUNK), jnp.int32),
        pltpu.VMEM((1, CHUNK), jnp.int32),
        pltpu.VMEM_SHARED((NPH, D), jnp.float32),
        pltpu.SemaphoreType.DMA,
        pltpu.SemaphoreType.DMA,
        pltpu.SemaphoreType.DMA,
        pltpu.SemaphoreType.DMA,
    ],
)(_edge_body)


def _div_body(acc_ref, out_ref):
    for h in range(2):
      for p in range(2):
        n = acc_ref[h, p, :RH, :H]
        d = acc_ref[h, p, :RH, H:]
        safe = jnp.where(d > 0.0, d, 1.0)
        out_ref[p * RH:(p + 1) * RH, h * H:(h + 1) * H] = jnp.where(
            d > 0.0, n / safe, 0.0)


def _divide(acc):
    return pl.pallas_call(
        _div_body,
        out_shape=jax.ShapeDtypeStruct((N, D), jnp.float32),
    )(acc)


def kernel(x, sender_idx, receiver_idx, type_idx, V_proj_sender,
           V_gate_sender, E_gate, B_message, B_gate_pre):
    cmb = _project(x, V_proj_sender, V_gate_sender, B_message,
                   B_gate_pre).reshape(2 * N, D)
    egs = E_gate.reshape(T, 2, H).transpose(1, 0, 2)
    egt = jnp.zeros((2, TP, H), jnp.float32).at[:, :T, :].set(egs)
    zer = jnp.zeros((NPH, D), jnp.float32)
    pad = EP - E
    sidx = jnp.concatenate(
        [sender_idx, jnp.zeros((pad,), jnp.int32)]).reshape(NCH, CHUNK)
    ridx = jnp.concatenate(
        [receiver_idx, jnp.full((pad,), N, jnp.int32)]).reshape(NCH, CHUNK)
    tidx = jnp.concatenate(
        [type_idx, jnp.zeros((pad,), jnp.int32)]).reshape(NCH, CHUNK)
    idx = jnp.stack([sidx, ridx, tidx], axis=1)
    acc = _edge_kernel(cmb, egt, idx, zer)
    return _divide(acc)


# speedup vs baseline: 8.2363x; 2.0848x over previous
import functools

import jax
import jax.numpy as jnp
from jax import lax
from jax.experimental import pallas as pl
from jax.experimental.pallas import tpu as pltpu
from jax.experimental.pallas import tpu_sc as plsc

N = 10000
D = 128
H = D // 2
E = 320000
T = 475
TP = 480
NS, L = 16, 16
CHUNK = 96
CPT = 216
EP = NS * CPT * CHUNK
NCH = NS * CPT
RH = N // 2
NPH = 5120
RPT = NPH // NS
TRASH = RH


def _proj_body(x_ref, vp_ref, vg_ref, bm_ref, bg_ref, out_ref):
    x = x_ref[...]
    xp = jnp.dot(x, vp_ref[...], preferred_element_type=jnp.float32)
    xg = jnp.dot(x, vg_ref[...], preferred_element_type=jnp.float32)
    xp = jnp.maximum(xp + bm_ref[...], 0.0)
    xg = xg + bg_ref[...]
    for h in range(2):
        out_ref[h, :, :H] = xp[:, h * H:(h + 1) * H]
        out_ref[h, :, H:] = xg[:, h * H:(h + 1) * H]


def _project(x, vp, vg, bm, bg):
    return pl.pallas_call(
        _proj_body,
        out_shape=jax.ShapeDtypeStruct((2, N, D), jnp.float32),
    )(x, vp, vg, bm.reshape(1, D), bg.reshape(1, D))


def _edge_body(cmb_hbm, egt_hbm, idx_hbm, zer_hbm,
               acc_out,
               idx0_v, idx1_v, egt_v, buf_v, sadj_v, radj_v, tadj_v,
               pends_v, pendr_v, pendt_v, acc_sh, semg, semi0, semi1):
    s = lax.axis_index("s")
    c = lax.axis_index("c")
    arows = pl.ds(s * RPT, RPT)
    cbase = s * CPT

    idxs = (idx0_v, idx1_v)
    semi = (semi0, semi1)
    nsl = H // L

    def fire_idx(k, slot):
        pltpu.async_copy(idx_hbm.at[cbase + k], idxs[slot], semi[slot])

    def wait_idx(slot, k):
        pltpu.make_async_copy(
            idx_hbm.at[cbase + k], idxs[slot], semi[slot]).wait()

    def compute_and_scatter():
        def blk(b, carry3):
            tvec = tadj_v[pl.ds(b * L, L)]
            for i0 in range(0, L, 2):
                rows = []
                for i in (i0, i0 + 1):
                    j = b * L + i
                    t = tvec[i]
                    g = [buf_v[j, pl.ds(H + l * L, L)] for l in range(nsl)]
                    eg = [egt_v[t, pl.ds(l * L, L)] for l in range(nsl)]
                    m = [buf_v[j, pl.ds(l * L, L)] for l in range(nsl)]
                    rows.append((j, g, eg, m))
                ev = [[jnp.exp(g[l] + eg[l]) for l in range(nsl)]
                      for (j, g, eg, m) in rows]
                for (j, g, eg, m), e in zip(rows, ev):
                    for l in range(nsl):
                        buf_v[j, pl.ds(H + l * L, L)] = e[l]
                        buf_v[j, pl.ds(l * L, L)] = m[l] * e[l]
            return carry3

        lax.fori_loop(0, CHUNK // L, blk, 0)
        pltpu.sync_copy(buf_v, acc_sh.at[radj_v.at[0]], add=True)

    def process_batch():
        for i in range(CHUNK // L):
            sl = pl.ds(i * L, L)
            sadj_v[sl] = pends_v[sl]
            radj_v[0, sl] = pendr_v[sl]
            tadj_v[sl] = pendt_v[sl]
        for i in range(CHUNK // L):
            sl = pl.ds(i * L, L)
            sh = pl.ds(CHUNK + i * L, L)
            pends_v[sl] = pends_v[sh]
            pendr_v[sl] = pendr_v[sh]
            pendt_v[sl] = pendt_v[sh]
        pltpu.async_copy(cmb_hbm.at[sadj_v], buf_v, semg).wait()
        compute_and_scatter()

    def phase(p, carry):
        soff = c * N
        roff = p * RH

        pltpu.sync_copy(egt_hbm.at[c], egt_v)
        pltpu.sync_copy(zer_hbm.at[arows], acc_sh.at[arows])
        plsc.subcore_barrier()

        fire_idx(0, 0)
        fire_idx(1, 1)

        def do_chunk(k, si, cnt):
            wait_idx(si, k)
            idx_v = idxs[si]
            cnew = cnt
            for l in range(CHUNK // L):
                sl = pl.ds(l * L, L)
                s16 = idx_v[0, sl] + soff
                rl = idx_v[1, sl] - roff
                t16 = idx_v[2, sl]
                msk = (rl >= 0) & (rl < RH)
                m32 = jnp.where(msk, jnp.full((L,), 1, jnp.int32),
                                jnp.zeros((L,), jnp.int32))
                cs = plsc.cumsum(m32)
                pos = cnew + cs - 1
                plsc.store_scatter(pends_v, [pos], s16, mask=msk)
                plsc.store_scatter(pendr_v, [pos], rl, mask=msk)
                plsc.store_scatter(pendt_v, [pos], t16, mask=msk)
                cnew = cnew + cs[L - 1]

            @pl.when(k + 2 < CPT)
            def _():
                fire_idx(k + 2, si)

            @pl.when(cnew >= CHUNK)
            def _b():
                process_batch()

            return jnp.where(cnew >= CHUNK, cnew - CHUNK, cnew)

        def chunk2(m, cnt):
            cnt = do_chunk(2 * m, 0, cnt)
            cnt = do_chunk(2 * m + 1, 1, cnt)
            return cnt

        cnt = lax.fori_loop(0, CPT // 2, chunk2, jnp.int32(0))

        trash_s = jnp.zeros((L,), jnp.int32) + soff
        trash_r = jnp.full((L,), TRASH, jnp.int32)
        trash_t = jnp.zeros((L,), jnp.int32)
        for i in range(CHUNK // L):
            base = pl.ds(cnt + i * L, L)
            pends_v[base] = trash_s
            pendr_v[base] = trash_r
            pendt_v[base] = trash_t
        process_batch()

        plsc.subcore_barrier()
        pltpu.sync_copy(acc_sh.at[arows], acc_out.at[c, p, arows])
        plsc.subcore_barrier()
        return carry

    lax.fori_loop(0, 2, phase, 0)


_edge_kernel = functools.partial(
    pl.kernel,
    out_type=jax.ShapeDtypeStruct((2, 2, NPH, D), jnp.float32),
    mesh=plsc.VectorSubcoreMesh(
        core_axis_name="c", subcore_axis_name="s", num_cores=2,
        num_subcores=NS),
    compiler_params=pltpu.CompilerParams(needs_layout_passes=False),
    scratch_types=[
        pltpu.VMEM((3, CHUNK), jnp.int32),
        pltpu.VMEM((3, CHUNK), jnp.int32),
        pltpu.VMEM((TP, H), jnp.float32),
        pltpu.VMEM((CHUNK, D), jnp.float32),
        pltpu.VMEM((CHUNK,), jnp.int32),
        pltpu.VMEM((1, CHUNK), jnp.int32),
        pltpu.VMEM((CHUNK,), jnp.int32),
        pltpu.VMEM((2 * CHUNK,), jnp.int32),
        pltpu.VMEM((2 * CHUNK,), jnp.int32),
        pltpu.VMEM((2 * CHUNK,), jnp.int32),
        pltpu.VMEM_SHARED((NPH, D), jnp.float32),
        pltpu.SemaphoreType.DMA,
        pltpu.SemaphoreType.DMA,
        pltpu.SemaphoreType.DMA,
    ],
)(_edge_body)


def _div_body(acc_ref, out_ref):
    for h in range(2):
      for p in range(2):
        n = acc_ref[h, p, :RH, :H]
        d = acc_ref[h, p, :RH, H:]
        safe = jnp.where(d > 0.0, d, 1.0)
        out_ref[p * RH:(p + 1) * RH, h * H:(h + 1) * H] = jnp.where(
            d > 0.0, n / safe, 0.0)


def _divide(acc):
    return pl.pallas_call(
        _div_body,
        out_shape=jax.ShapeDtypeStruct((N, D), jnp.float32),
    )(acc)


def kernel(x, sender_idx, receiver_idx, type_idx, V_proj_sender,
           V_gate_sender, E_gate, B_message, B_gate_pre):
    cmb = _project(x, V_proj_sender, V_gate_sender, B_message,
                   B_gate_pre).reshape(2 * N, D)
    egs = E_gate.reshape(T, 2, H).transpose(1, 0, 2)
    egt = jnp.zeros((2, TP, H), jnp.float32).at[:, :T, :].set(egs)
    zer = jnp.zeros((NPH, D), jnp.float32)
    pad = EP - E
    sidx = jnp.concatenate(
        [sender_idx, jnp.zeros((pad,), jnp.int32)]).reshape(NCH, CHUNK)
    ridx = jnp.concatenate(
        [receiver_idx, jnp.full((pad,), N, jnp.int32)]).reshape(NCH, CHUNK)
    tidx = jnp.concatenate(
        [type_idx, jnp.zeros((pad,), jnp.int32)]).reshape(NCH, CHUNK)
    idx = jnp.stack([sidx, ridx, tidx], axis=1)
    acc = _edge_kernel(cmb, egt, idx, zer)
    return _divide(acc)


# speedup vs baseline: 8.6352x; 1.0484x over previous
import functools

import jax
import jax.numpy as jnp
from jax import lax
from jax.experimental import pallas as pl
from jax.experimental.pallas import tpu as pltpu
from jax.experimental.pallas import tpu_sc as plsc

N = 10000
D = 128
H = D // 2
E = 320000
T = 475
TP = 480
NS, L = 16, 16
CHUNK = 128
CPT = 160
EP = NS * CPT * CHUNK
NCH = NS * CPT
RH = N // 2
NPH = 5120
RPT = NPH // NS
TRASH = RH


def _proj_body(x_ref, vp_ref, vg_ref, bm_ref, bg_ref, out_ref):
    x = x_ref[...]
    xp = jnp.dot(x, vp_ref[...], preferred_element_type=jnp.float32)
    xg = jnp.dot(x, vg_ref[...], preferred_element_type=jnp.float32)
    xp = jnp.maximum(xp + bm_ref[...], 0.0)
    xg = xg + bg_ref[...]
    for h in range(2):
        out_ref[h, :, :H] = xp[:, h * H:(h + 1) * H]
        out_ref[h, :, H:] = xg[:, h * H:(h + 1) * H]


def _project(x, vp, vg, bm, bg):
    return pl.pallas_call(
        _proj_body,
        out_shape=jax.ShapeDtypeStruct((2, N, D), jnp.float32),
    )(x, vp, vg, bm.reshape(1, D), bg.reshape(1, D))


def _edge_body(cmb_hbm, egt_hbm, idx_hbm, zer_hbm,
               acc_out,
               idx0_v, idx1_v, egt_v, buf_v, sadj_v, radj_v, tadj_v,
               pends_v, pendr_v, pendt_v, acc_sh, semg, semi0, semi1):
    s = lax.axis_index("s")
    c = lax.axis_index("c")
    arows = pl.ds(s * RPT, RPT)
    cbase = s * CPT

    idxs = (idx0_v, idx1_v)
    semi = (semi0, semi1)
    nsl = H // L

    def fire_idx(k, slot):
        pltpu.async_copy(idx_hbm.at[cbase + k], idxs[slot], semi[slot])

    def wait_idx(slot, k):
        pltpu.make_async_copy(
            idx_hbm.at[cbase + k], idxs[slot], semi[slot]).wait()

    def compute_and_scatter():
        def blk(b, carry3):
            tvec = tadj_v[pl.ds(b * L, L)]
            for i0 in range(0, L, 2):
                rows = []
                for i in (i0, i0 + 1):
                    j = b * L + i
                    t = tvec[i]
                    g = [buf_v[j, pl.ds(H + l * L, L)] for l in range(nsl)]
                    eg = [egt_v[t, pl.ds(l * L, L)] for l in range(nsl)]
                    m = [buf_v[j, pl.ds(l * L, L)] for l in range(nsl)]
                    rows.append((j, g, eg, m))
                ev = [[jnp.exp(g[l] + eg[l]) for l in range(nsl)]
                      for (j, g, eg, m) in rows]
                for (j, g, eg, m), e in zip(rows, ev):
                    for l in range(nsl):
                        buf_v[j, pl.ds(H + l * L, L)] = e[l]
                        buf_v[j, pl.ds(l * L, L)] = m[l] * e[l]
            return carry3

        lax.fori_loop(0, CHUNK // L, blk, 0)
        pltpu.sync_copy(buf_v, acc_sh.at[radj_v.at[0]], add=True)

    def process_batch():
        for i in range(CHUNK // L):
            sl = pl.ds(i * L, L)
            sadj_v[sl] = pends_v[sl]
            radj_v[0, sl] = pendr_v[sl]
            tadj_v[sl] = pendt_v[sl]
        for i in range(CHUNK // L):
            sl = pl.ds(i * L, L)
            sh = pl.ds(CHUNK + i * L, L)
            pends_v[sl] = pends_v[sh]
            pendr_v[sl] = pendr_v[sh]
            pendt_v[sl] = pendt_v[sh]
        pltpu.async_copy(cmb_hbm.at[sadj_v], buf_v, semg).wait()
        compute_and_scatter()

    def phase(p, carry):
        soff = c * N
        roff = p * RH

        pltpu.sync_copy(egt_hbm.at[c], egt_v)
        pltpu.sync_copy(zer_hbm.at[arows], acc_sh.at[arows])
        plsc.subcore_barrier()

        fire_idx(0, 0)
        fire_idx(1, 1)

        def do_chunk(k, si, cnt):
            wait_idx(si, k)
            idx_v = idxs[si]
            cnew = cnt
            for l in range(CHUNK // L):
                sl = pl.ds(l * L, L)
                s16 = idx_v[0, sl] + soff
                rl = idx_v[1, sl] - roff
                t16 = idx_v[2, sl]
                msk = (rl >= 0) & (rl < RH)
                m32 = jnp.where(msk, jnp.full((L,), 1, jnp.int32),
                                jnp.zeros((L,), jnp.int32))
                cs = plsc.cumsum(m32)
                pos = cnew + cs - 1
                plsc.store_scatter(pends_v, [pos], s16, mask=msk)
                plsc.store_scatter(pendr_v, [pos], rl, mask=msk)
                plsc.store_scatter(pendt_v, [pos], t16, mask=msk)
                cnew = cnew + cs[L - 1]

            @pl.when(k + 2 < CPT)
            def _():
                fire_idx(k + 2, si)

            @pl.when(cnew >= CHUNK)
            def _b():
                process_batch()

            return jnp.where(cnew >= CHUNK, cnew - CHUNK, cnew)

        def chunk2(m, cnt):
            cnt = do_chunk(2 * m, 0, cnt)
            cnt = do_chunk(2 * m + 1, 1, cnt)
            return cnt

        cnt = lax.fori_loop(0, CPT // 2, chunk2, jnp.int32(0))

        trash_s = jnp.zeros((L,), jnp.int32) + soff
        trash_r = jnp.full((L,), TRASH, jnp.int32)
        trash_t = jnp.zeros((L,), jnp.int32)
        for i in range(CHUNK // L):
            base = pl.ds(cnt + i * L, L)
            pends_v[base] = trash_s
            pendr_v[base] = trash_r
            pendt_v[base] = trash_t
        process_batch()

        plsc.subcore_barrier()
        pltpu.sync_copy(acc_sh.at[arows], acc_out.at[c, p, arows])
        plsc.subcore_barrier()
        return carry

    lax.fori_loop(0, 2, phase, 0)


_edge_kernel = functools.partial(
    pl.kernel,
    out_type=jax.ShapeDtypeStruct((2, 2, NPH, D), jnp.float32),
    mesh=plsc.VectorSubcoreMesh(
        core_axis_name="c", subcore_axis_name="s", num_cores=2,
        num_subcores=NS),
    compiler_params=pltpu.CompilerParams(needs_layout_passes=False),
    scratch_types=[
        pltpu.VMEM((3, CHUNK), jnp.int32),
        pltpu.VMEM((3, CHUNK), jnp.int32),
        pltpu.VMEM((TP, H), jnp.float32),
        pltpu.VMEM((CHUNK, D), jnp.float32),
        pltpu.VMEM((CHUNK,), jnp.int32),
        pltpu.VMEM((1, CHUNK), jnp.int32),
        pltpu.VMEM((CHUNK,), jnp.int32),
        pltpu.VMEM((2 * CHUNK,), jnp.int32),
        pltpu.VMEM((2 * CHUNK,), jnp.int32),
        pltpu.VMEM((2 * CHUNK,), jnp.int32),
        pltpu.VMEM_SHARED((NPH, D), jnp.float32),
        pltpu.SemaphoreType.DMA,
        pltpu.SemaphoreType.DMA,
        pltpu.SemaphoreType.DMA,
    ],
)(_edge_body)


def _div_body(acc_ref, out_ref):
    for h in range(2):
      for p in range(2):
        n = acc_ref[h, p, :RH, :H]
        d = acc_ref[h, p, :RH, H:]
        safe = jnp.where(d > 0.0, d, 1.0)
        out_ref[p * RH:(p + 1) * RH, h * H:(h + 1) * H] = jnp.where(
            d > 0.0, n / safe, 0.0)


def _divide(acc):
    return pl.pallas_call(
        _div_body,
        out_shape=jax.ShapeDtypeStruct((N, D), jnp.float32),
    )(acc)


def kernel(x, sender_idx, receiver_idx, type_idx, V_proj_sender,
           V_gate_sender, E_gate, B_message, B_gate_pre):
    cmb = _project(x, V_proj_sender, V_gate_sender, B_message,
                   B_gate_pre).reshape(2 * N, D)
    egs = E_gate.reshape(T, 2, H).transpose(1, 0, 2)
    egt = jnp.zeros((2, TP, H), jnp.float32).at[:, :T, :].set(egs)
    zer = jnp.zeros((NPH, D), jnp.float32)
    pad = EP - E
    sidx = jnp.concatenate(
        [sender_idx, jnp.zeros((pad,), jnp.int32)]).reshape(NCH, CHUNK)
    ridx = jnp.concatenate(
        [receiver_idx, jnp.full((pad,), N, jnp.int32)]).reshape(NCH, CHUNK)
    tidx = jnp.concatenate(
        [type_idx, jnp.zeros((pad,), jnp.int32)]).reshape(NCH, CHUNK)
    idx = jnp.stack([sidx, ridx, tidx], axis=1)
    acc = _edge_kernel(cmb, egt, idx, zer)
    return _divide(acc)


# speedup vs baseline: 9.5964x; 1.1113x over previous
import functools

import jax
import jax.numpy as jnp
from jax import lax
from jax.experimental import pallas as pl
from jax.experimental.pallas import tpu as pltpu
from jax.experimental.pallas import tpu_sc as plsc

N = 10000
D = 128
H = D // 2
E = 320000
T = 475
TP = 480
NS, L = 16, 16
CHUNK = 128
CPT = 160
EP = NS * CPT * CHUNK
NCH = NS * CPT
RH = N // 2
NPH = 5120
RPT = NPH // NS
TRASH = RH


def _proj_body(x_ref, vp_ref, vg_ref, bm_ref, bg_ref, egt_ref, out_ref,
               ege_ref):
    x = x_ref[...]
    xp = jnp.dot(x, vp_ref[...], preferred_element_type=jnp.float32)
    xg = jnp.dot(x, vg_ref[...], preferred_element_type=jnp.float32)
    xp = jnp.maximum(xp + bm_ref[...], 0.0)
    xg = jnp.exp(xg + bg_ref[...])
    for h in range(2):
        out_ref[h, :, :H] = xp[:, h * H:(h + 1) * H]
        out_ref[h, :, H:] = xg[:, h * H:(h + 1) * H]
    ege_ref[...] = jnp.exp(egt_ref[...])


def _project(x, vp, vg, bm, bg, egt):
    return pl.pallas_call(
        _proj_body,
        out_shape=[
            jax.ShapeDtypeStruct((2, N, D), jnp.float32),
            jax.ShapeDtypeStruct((2, TP, H), jnp.float32),
        ],
    )(x, vp, vg, bm.reshape(1, D), bg.reshape(1, D), egt)


def _edge_body(cmb_hbm, egt_hbm, idx_hbm, zer_hbm,
               acc_out,
               idx0_v, idx1_v, egt_v, buf_v, sadj_v, radj_v, tadj_v,
               pends_v, pendr_v, pendt_v, acc_sh, semg, semi0, semi1):
    s = lax.axis_index("s")
    c = lax.axis_index("c")
    arows = pl.ds(s * RPT, RPT)
    cbase = s * CPT

    idxs = (idx0_v, idx1_v)
    semi = (semi0, semi1)
    nsl = H // L

    def fire_idx(k, slot):
        pltpu.async_copy(idx_hbm.at[cbase + k], idxs[slot], semi[slot])

    def wait_idx(slot, k):
        pltpu.make_async_copy(
            idx_hbm.at[cbase + k], idxs[slot], semi[slot]).wait()

    def compute_and_scatter():
        def blk(b, carry3):
            tvec = tadj_v[pl.ds(b * L, L)]
            for i0 in range(0, L, 2):
                rows = []
                for i in (i0, i0 + 1):
                    j = b * L + i
                    t = tvec[i]
                    g = [buf_v[j, pl.ds(H + l * L, L)] for l in range(nsl)]
                    eg = [egt_v[t, pl.ds(l * L, L)] for l in range(nsl)]
                    m = [buf_v[j, pl.ds(l * L, L)] for l in range(nsl)]
                    rows.append((j, g, eg, m))
                ev = [[g[l] * eg[l] for l in range(nsl)]
                      for (j, g, eg, m) in rows]
                for (j, g, eg, m), e in zip(rows, ev):
                    for l in range(nsl):
                        buf_v[j, pl.ds(H + l * L, L)] = e[l]
                        buf_v[j, pl.ds(l * L, L)] = m[l] * e[l]
            return carry3

        lax.fori_loop(0, CHUNK // L, blk, 0)
        pltpu.sync_copy(buf_v, acc_sh.at[radj_v.at[0]], add=True)

    def process_batch():
        for i in range(CHUNK // L):
            sl = pl.ds(i * L, L)
            sadj_v[sl] = pends_v[sl]
            radj_v[0, sl] = pendr_v[sl]
            tadj_v[sl] = pendt_v[sl]
        for i in range(CHUNK // L):
            sl = pl.ds(i * L, L)
            sh = pl.ds(CHUNK + i * L, L)
            pends_v[sl] = pends_v[sh]
            pendr_v[sl] = pendr_v[sh]
            pendt_v[sl] = pendt_v[sh]
        pltpu.async_copy(cmb_hbm.at[sadj_v], buf_v, semg).wait()
        compute_and_scatter()

    def phase(p, carry):
        soff = c * N
        roff = p * RH

        pltpu.sync_copy(egt_hbm.at[c], egt_v)
        pltpu.sync_copy(zer_hbm.at[arows], acc_sh.at[arows])
        plsc.subcore_barrier()

        fire_idx(0, 0)
        fire_idx(1, 1)

        def do_chunk(k, si, cnt):
            wait_idx(si, k)
            idx_v = idxs[si]
            cnew = cnt
            for l in range(CHUNK // L):
                sl = pl.ds(l * L, L)
                s16 = idx_v[0, sl] + soff
                rl = idx_v[1, sl] - roff
                t16 = idx_v[2, sl]
                msk = (rl >= 0) & (rl < RH)
                m32 = jnp.where(msk, jnp.full((L,), 1, jnp.int32),
                                jnp.zeros((L,), jnp.int32))
                cs = plsc.cumsum(m32)
                pos = cnew + cs - 1
                plsc.store_scatter(pends_v, [pos], s16, mask=msk)
                plsc.store_scatter(pendr_v, [pos], rl, mask=msk)
                plsc.store_scatter(pendt_v, [pos], t16, mask=msk)
                cnew = cnew + cs[L - 1]

            @pl.when(k + 2 < CPT)
            def _():
                fire_idx(k + 2, si)

            @pl.when(cnew >= CHUNK)
            def _b():
                process_batch()

            return jnp.where(cnew >= CHUNK, cnew - CHUNK, cnew)

        def chunk2(m, cnt):
            cnt = do_chunk(2 * m, 0, cnt)
            cnt = do_chunk(2 * m + 1, 1, cnt)
            return cnt

        cnt = lax.fori_loop(0, CPT // 2, chunk2, jnp.int32(0))

        trash_s = jnp.zeros((L,), jnp.int32) + soff
        trash_r = jnp.full((L,), TRASH, jnp.int32)
        trash_t = jnp.zeros((L,), jnp.int32)
        for i in range(CHUNK // L):
            base = pl.ds(cnt + i * L, L)
            pends_v[base] = trash_s
            pendr_v[base] = trash_r
            pendt_v[base] = trash_t
        process_batch()

        plsc.subcore_barrier()
        pltpu.sync_copy(acc_sh.at[arows], acc_out.at[c, p, arows])
        plsc.subcore_barrier()
        return carry

    lax.fori_loop(0, 2, phase, 0)


_edge_kernel = functools.partial(
    pl.kernel,
    out_type=jax.ShapeDtypeStruct((2, 2, NPH, D), jnp.float32),
    mesh=plsc.VectorSubcoreMesh(
        core_axis_name="c", subcore_axis_name="s", num_cores=2,
        num_subcores=NS),
    compiler_params=pltpu.CompilerParams(needs_layout_passes=False),
    scratch_types=[
        pltpu.VMEM((3, CHUNK), jnp.int32),
        pltpu.VMEM((3, CHUNK), jnp.int32),
        pltpu.VMEM((TP, H), jnp.float32),
        pltpu.VMEM((CHUNK, D), jnp.float32),
        pltpu.VMEM((CHUNK,), jnp.int32),
        pltpu.VMEM((1, CHUNK), jnp.int32),
        pltpu.VMEM((CHUNK,), jnp.int32),
        pltpu.VMEM((2 * CHUNK,), jnp.int32),
        pltpu.VMEM((2 * CHUNK,), jnp.int32),
        pltpu.VMEM((2 * CHUNK,), jnp.int32),
        pltpu.VMEM_SHARED((NPH, D), jnp.float32),
        pltpu.SemaphoreType.DMA,
        pltpu.SemaphoreType.DMA,
        pltpu.SemaphoreType.DMA,
    ],
)(_edge_body)


def _div_body(acc_ref, out_ref):
    for h in range(2):
      for p in range(2):
        n = acc_ref[h, p, :RH, :H]
        d = acc_ref[h, p, :RH, H:]
        safe = jnp.where(d > 0.0, d, 1.0)
        out_ref[p * RH:(p + 1) * RH, h * H:(h + 1) * H] = jnp.where(
            d > 0.0, n / safe, 0.0)


def _divide(acc):
    return pl.pallas_call(
        _div_body,
        out_shape=jax.ShapeDtypeStruct((N, D), jnp.float32),
    )(acc)


def kernel(x, sender_idx, receiver_idx, type_idx, V_proj_sender,
           V_gate_sender, E_gate, B_message, B_gate_pre):
    egs = E_gate.reshape(T, 2, H).transpose(1, 0, 2)
    egt0 = jnp.zeros((2, TP, H), jnp.float32).at[:, :T, :].set(egs)
    cmb, egt = _project(x, V_proj_sender, V_gate_sender, B_message,
                        B_gate_pre, egt0)
    cmb = cmb.reshape(2 * N, D)
    zer = jnp.zeros((NPH, D), jnp.float32)
    pad = EP - E
    sidx = jnp.concatenate(
        [sender_idx, jnp.zeros((pad,), jnp.int32)]).reshape(NCH, CHUNK)
    ridx = jnp.concatenate(
        [receiver_idx, jnp.full((pad,), N, jnp.int32)]).reshape(NCH, CHUNK)
    tidx = jnp.concatenate(
        [type_idx, jnp.zeros((pad,), jnp.int32)]).reshape(NCH, CHUNK)
    idx = jnp.stack([sidx, ridx, tidx], axis=1)
    acc = _edge_kernel(cmb, egt, idx, zer)
    return _divide(acc)


# speedup vs baseline: 9.6609x; 1.0067x over previous
import functools

import jax
import jax.numpy as jnp
from jax import lax
from jax.experimental import pallas as pl
from jax.experimental.pallas import tpu as pltpu
from jax.experimental.pallas import tpu_sc as plsc

N = 10000
D = 128
H = D // 2
E = 320000
T = 475
TP = 480
NS, L = 16, 16
CHUNK = 96
CPT = 216
EP = NS * CPT * CHUNK
NCH = NS * CPT
RH = N // 2
NPH = 5120
RPT = NPH // NS
TRASH = RH


def _proj_body(x_ref, vp_ref, vg_ref, bm_ref, bg_ref, egt_ref, out_ref,
               ege_ref):
    x = x_ref[...]
    xp = jnp.dot(x, vp_ref[...], preferred_element_type=jnp.float32)
    xg = jnp.dot(x, vg_ref[...], preferred_element_type=jnp.float32)
    xp = jnp.maximum(xp + bm_ref[...], 0.0)
    xg = jnp.exp(xg + bg_ref[...])
    for h in range(2):
        out_ref[h, :, :H] = xp[:, h * H:(h + 1) * H]
        out_ref[h, :, H:] = xg[:, h * H:(h + 1) * H]
    ege_ref[...] = jnp.exp(egt_ref[...])


def _project(x, vp, vg, bm, bg, egt):
    return pl.pallas_call(
        _proj_body,
        out_shape=[
            jax.ShapeDtypeStruct((2, N, D), jnp.float32),
            jax.ShapeDtypeStruct((2, TP, H), jnp.float32),
        ],
    )(x, vp, vg, bm.reshape(1, D), bg.reshape(1, D), egt)


def _edge_body(cmb_hbm, egt_hbm, idx_hbm, zer_hbm,
               acc_out,
               idx0_v, idx1_v, egt_v, buf0_v, buf1_v, sadj0_v, sadj1_v,
               radj0_v, radj1_v, tadj0_v, tadj1_v,
               pends_v, pendr_v, pendt_v, acc_sh, semg0, semg1, semi0,
               semi1):
    s = lax.axis_index("s")
    c = lax.axis_index("c")
    arows = pl.ds(s * RPT, RPT)
    cbase = s * CPT

    idxs = (idx0_v, idx1_v)
    semi = (semi0, semi1)
    bufs = (buf0_v, buf1_v)
    sadjs = (sadj0_v, sadj1_v)
    radjs = (radj0_v, radj1_v)
    tadjs = (tadj0_v, tadj1_v)
    semg = (semg0, semg1)
    nsl = H // L

    def fire_idx(k, slot):
        pltpu.async_copy(idx_hbm.at[cbase + k], idxs[slot], semi[slot])

    def wait_idx(slot, k):
        pltpu.make_async_copy(
            idx_hbm.at[cbase + k], idxs[slot], semi[slot]).wait()

    def compute_and_scatter(slot):
        buf_v = bufs[slot]
        tadj_v = tadjs[slot]

        def blk(b, carry3):
            tvec = tadj_v[pl.ds(b * L, L)]
            for i0 in range(0, L, 2):
                rows = []
                for i in (i0, i0 + 1):
                    j = b * L + i
                    t = tvec[i]
                    g = [buf_v[j, pl.ds(H + l * L, L)] for l in range(nsl)]
                    eg = [egt_v[t, pl.ds(l * L, L)] for l in range(nsl)]
                    m = [buf_v[j, pl.ds(l * L, L)] for l in range(nsl)]
                    rows.append((j, g, eg, m))
                ev = [[g[l] * eg[l] for l in range(nsl)]
                      for (j, g, eg, m) in rows]
                for (j, g, eg, m), e in zip(rows, ev):
                    for l in range(nsl):
                        buf_v[j, pl.ds(H + l * L, L)] = e[l]
                        buf_v[j, pl.ds(l * L, L)] = m[l] * e[l]
            return carry3

        lax.fori_loop(0, CHUNK // L, blk, 0)
        pltpu.sync_copy(buf_v, acc_sh.at[radjs[slot].at[0]], add=True)

    def flush_fire(slot):
        for i in range(CHUNK // L):
            sl = pl.ds(i * L, L)
            sadjs[slot][sl] = pends_v[sl]
            radjs[slot][0, sl] = pendr_v[sl]
            tadjs[slot][sl] = pendt_v[sl]
        for i in range(CHUNK // L):
            sl = pl.ds(i * L, L)
            sh = pl.ds(CHUNK + i * L, L)
            pends_v[sl] = pends_v[sh]
            pendr_v[sl] = pendr_v[sh]
            pendt_v[sl] = pendt_v[sh]
        pltpu.async_copy(cmb_hbm.at[sadjs[slot]], bufs[slot], semg[slot])

    def drain(slot):
        pltpu.make_async_copy(
            cmb_hbm.at[sadjs[slot]], bufs[slot], semg[slot]).wait()
        compute_and_scatter(slot)

    def phase(p, carry):
        soff = c * N
        roff = p * RH

        pltpu.sync_copy(egt_hbm.at[c], egt_v)
        pltpu.sync_copy(zer_hbm.at[arows], acc_sh.at[arows])
        plsc.subcore_barrier()

        fire_idx(0, 0)
        fire_idx(1, 1)

        def do_chunk(k, si, cnt, nb):
            wait_idx(si, k)
            idx_v = idxs[si]
            cnew = cnt
            for l in range(CHUNK // L):
                sl = pl.ds(l * L, L)
                s16 = idx_v[0, sl] + soff
                rl = idx_v[1, sl] - roff
                t16 = idx_v[2, sl]
                msk = (rl >= 0) & (rl < RH)
                m32 = jnp.where(msk, jnp.full((L,), 1, jnp.int32),
                                jnp.zeros((L,), jnp.int32))
                cs = plsc.cumsum(m32)
                pos = cnew + cs - 1
                plsc.store_scatter(pends_v, [pos], s16, mask=msk)
                plsc.store_scatter(pendr_v, [pos], rl, mask=msk)
                plsc.store_scatter(pendt_v, [pos], t16, mask=msk)
                cnew = cnew + cs[L - 1]

            @pl.when(k + 2 < CPT)
            def _():
                fire_idx(k + 2, si)

            trig = cnew >= CHUNK
            par = nb % 2

            @pl.when(trig & (par == 0))
            def _b0():
                @pl.when(nb > 0)
                def _d1():
                    drain(1)
                flush_fire(0)

            @pl.when(trig & (par == 1))
            def _b1():
                drain(0)
                flush_fire(1)

            cnew = jnp.where(trig, cnew - CHUNK, cnew)
            return cnew, jnp.where(trig, nb + 1, nb)

        def chunk2(m, carry2):
            cnt, nb = carry2
            cnt, nb = do_chunk(2 * m, 0, cnt, nb)
            cnt, nb = do_chunk(2 * m + 1, 1, cnt, nb)
            return cnt, nb

        cnt, nb = lax.fori_loop(0, CPT // 2, chunk2,
                                (jnp.int32(0), jnp.int32(0)))

        trash_s = jnp.zeros((L,), jnp.int32) + soff
        trash_r = jnp.full((L,), TRASH, jnp.int32)
        trash_t = jnp.zeros((L,), jnp.int32)
        for i in range(CHUNK // L):
            base = pl.ds(cnt + i * L, L)
            pends_v[base] = trash_s
            pendr_v[base] = trash_r
            pendt_v[base] = trash_t

        @pl.when(nb % 2 == 0)
        def _t0():
            @pl.when(nb > 0)
            def _td1():
                drain(1)
            flush_fire(0)
            drain(0)

        @pl.when(nb % 2 == 1)
        def _t1():
            drain(0)
            flush_fire(1)
            drain(1)

        plsc.subcore_barrier()
        pltpu.sync_copy(acc_sh.at[arows], acc_out.at[c, p, arows])
        plsc.subcore_barrier()
        return carry

    lax.fori_loop(0, 2, phase, 0)


_edge_kernel = functools.partial(
    pl.kernel,
    out_type=jax.ShapeDtypeStruct((2, 2, NPH, D), jnp.float32),
    mesh=plsc.VectorSubcoreMesh(
        core_axis_name="c", subcore_axis_name="s", num_cores=2,
        num_subcores=NS),
    compiler_params=pltpu.CompilerParams(needs_layout_passes=False),
    scratch_types=[
        pltpu.VMEM((3, CHUNK), jnp.int32),
        pltpu.VMEM((3, CHUNK), jnp.int32),
        pltpu.VMEM((TP, H), jnp.float32),
        pltpu.VMEM((CHUNK, D), jnp.float32),
        pltpu.VMEM((CHUNK, D), jnp.float32),
        pltpu.VMEM((CHUNK,), jnp.int32),
        pltpu.VMEM((CHUNK,), jnp.int32),
        pltpu.VMEM((1, CHUNK), jnp.int32),
        pltpu.VMEM((1, CHUNK), jnp.int32),
        pltpu.VMEM((CHUNK,), jnp.int32),
        pltpu.VMEM((CHUNK,), jnp.int32),
        pltpu.VMEM((2 * CHUNK,), jnp.int32),
        pltpu.VMEM((2 * CHUNK,), jnp.int32),
        pltpu.VMEM((2 * CHUNK,), jnp.int32),
        pltpu.VMEM_SHARED((NPH, D), jnp.float32),
        pltpu.SemaphoreType.DMA,
        pltpu.SemaphoreType.DMA,
        pltpu.SemaphoreType.DMA,
        pltpu.SemaphoreType.DMA,
    ],
)(_edge_body)


def _div_body(acc_ref, out_ref):
    for h in range(2):
      for p in range(2):
        n = acc_ref[h, p, :RH, :H]
        d = acc_ref[h, p, :RH, H:]
        safe = jnp.where(d > 0.0, d, 1.0)
        out_ref[p * RH:(p + 1) * RH, h * H:(h + 1) * H] = jnp.where(
            d > 0.0, n / safe, 0.0)


def _divide(acc):
    return pl.pallas_call(
        _div_body,
        out_shape=jax.ShapeDtypeStruct((N, D), jnp.float32),
    )(acc)


def kernel(x, sender_idx, receiver_idx, type_idx, V_proj_sender,
           V_gate_sender, E_gate, B_message, B_gate_pre):
    egs = E_gate.reshape(T, 2, H).transpose(1, 0, 2)
    egt0 = jnp.zeros((2, TP, H), jnp.float32).at[:, :T, :].set(egs)
    cmb, egt = _project(x, V_proj_sender, V_gate_sender, B_message,
                        B_gate_pre, egt0)
    cmb = cmb.reshape(2 * N, D)
    zer = jnp.zeros((NPH, D), jnp.float32)
    pad = EP - E
    sidx = jnp.concatenate(
        [sender_idx, jnp.zeros((pad,), jnp.int32)]).reshape(NCH, CHUNK)
    ridx = jnp.concatenate(
        [receiver_idx, jnp.full((pad,), N, jnp.int32)]).reshape(NCH, CHUNK)
    tidx = jnp.concatenate(
        [type_idx, jnp.zeros((pad,), jnp.int32)]).reshape(NCH, CHUNK)
    idx = jnp.stack([sidx, ridx, tidx], axis=1)
    acc = _edge_kernel(cmb, egt, idx, zer)
    return _divide(acc)


# speedup vs baseline: 13.4374x; 1.3909x over previous
import functools

import jax
import jax.numpy as jnp
from jax import lax
from jax.experimental import pallas as pl
from jax.experimental.pallas import tpu as pltpu
from jax.experimental.pallas import tpu_sc as plsc

N = 10000
D = 128
H = D // 2
E = 320000
T = 475
TP = 480
NS, L = 16, 16
CHUNK = 96
CPT = 216
EP = NS * CPT * CHUNK
NCH = NS * CPT
RH = N // 2
NPH = 5120
RPT = NPH // NS
TRASH = RH


def _proj_body(x_ref, vp_ref, vg_ref, bm_ref, bg_ref, egt_ref, out_ref,
               ege_ref):
    x = x_ref[...]
    xp = jnp.dot(x, vp_ref[...], preferred_element_type=jnp.float32)
    xg = jnp.dot(x, vg_ref[...], preferred_element_type=jnp.float32)
    xp = jnp.maximum(xp + bm_ref[...], 0.0)
    xg = jnp.exp(xg + bg_ref[...])
    for h in range(2):
        out_ref[h, :, :H] = xp[:, h * H:(h + 1) * H]
        out_ref[h, :, H:] = xg[:, h * H:(h + 1) * H]
    ege_ref[...] = jnp.exp(egt_ref[...])


def _project(x, vp, vg, bm, bg, egt):
    return pl.pallas_call(
        _proj_body,
        out_shape=[
            jax.ShapeDtypeStruct((2, N, D), jnp.float32),
            jax.ShapeDtypeStruct((2, TP, H), jnp.float32),
        ],
    )(x, vp, vg, bm.reshape(1, D), bg.reshape(1, D), egt)


def _edge_body(cmb_hbm, egt_hbm, idx_hbm, zer_hbm,
               acc_out,
               idx0_v, idx1_v, egt_v, buf0_v, buf1_v, sadj0_v, sadj1_v,
               radj0_v, radj1_v, tadj0_v, tadj1_v,
               pends_v, pendr_v, pendt_v, acc_sh, semg0, semg1, semi0,
               semi1):
    s = lax.axis_index("s")
    c = lax.axis_index("c")
    arows = pl.ds(s * RPT, RPT)
    cbase = s * CPT

    idxs = (idx0_v, idx1_v)
    semi = (semi0, semi1)
    bufs = (buf0_v, buf1_v)
    sadjs = (sadj0_v, sadj1_v)
    radjs = (radj0_v, radj1_v)
    tadjs = (tadj0_v, tadj1_v)
    semg = (semg0, semg1)
    nsl = H // L

    def fire_idx(k, slot):
        pltpu.async_copy(idx_hbm.at[cbase + k], idxs[slot], semi[slot])

    def wait_idx(slot, k):
        pltpu.make_async_copy(
            idx_hbm.at[cbase + k], idxs[slot], semi[slot]).wait()

    def compute_and_scatter(slot):
        buf_v = bufs[slot]
        tadj_v = tadjs[slot]

        def blk(b, carry3):
            tvec = tadj_v[pl.ds(b * L, L)]
            for i0 in range(0, L, 2):
                rows = []
                for i in (i0, i0 + 1):
                    j = b * L + i
                    t = tvec[i]
                    g = [buf_v[j, pl.ds(H + l * L, L)] for l in range(nsl)]
                    eg = [egt_v[t, pl.ds(l * L, L)] for l in range(nsl)]
                    m = [buf_v[j, pl.ds(l * L, L)] for l in range(nsl)]
                    rows.append((j, g, eg, m))
                ev = [[g[l] * eg[l] for l in range(nsl)]
                      for (j, g, eg, m) in rows]
                for (j, g, eg, m), e in zip(rows, ev):
                    for l in range(nsl):
                        buf_v[j, pl.ds(H + l * L, L)] = e[l]
                        buf_v[j, pl.ds(l * L, L)] = m[l] * e[l]
            return carry3

        lax.fori_loop(0, CHUNK // L, blk, 0)
        pltpu.sync_copy(buf_v, acc_sh.at[radjs[slot].at[0]], add=True)

    def flush_fire(slot):
        for i in range(CHUNK // L):
            sl = pl.ds(i * L, L)
            sadjs[slot][sl] = pends_v[sl]
            radjs[slot][0, sl] = pendr_v[sl]
            tadjs[slot][sl] = pendt_v[sl]
        for i in range(CHUNK // L):
            sl = pl.ds(i * L, L)
            sh = pl.ds(CHUNK + i * L, L)
            pends_v[sl] = pends_v[sh]
            pendr_v[sl] = pendr_v[sh]
            pendt_v[sl] = pendt_v[sh]
        pltpu.async_copy(cmb_hbm.at[sadjs[slot]], bufs[slot], semg[slot])

    def drain(slot):
        pltpu.make_async_copy(
            cmb_hbm.at[sadjs[slot]], bufs[slot], semg[slot]).wait()
        compute_and_scatter(slot)

    def phase(p, carry):
        soff = c * N
        roff = p * RH

        pltpu.sync_copy(egt_hbm.at[c], egt_v)
        pltpu.sync_copy(zer_hbm.at[arows], acc_sh.at[arows])
        plsc.subcore_barrier()

        fire_idx(0, 0)
        fire_idx(1, 1)

        def do_chunk(k, si, cnt, nb):
            wait_idx(si, k)
            idx_v = idxs[si]
            cnew = cnt
            for l in range(CHUNK // L):
                sl = pl.ds(l * L, L)
                s16 = idx_v[0, sl] + soff
                rl = idx_v[1, sl] - roff
                t16 = idx_v[2, sl]
                msk = (rl >= 0) & (rl < RH)
                m32 = jnp.where(msk, jnp.full((L,), 1, jnp.int32),
                                jnp.zeros((L,), jnp.int32))
                cs = plsc.cumsum(m32)
                pos = cnew + cs - 1
                plsc.store_scatter(pends_v, [pos], s16, mask=msk)
                plsc.store_scatter(pendr_v, [pos], rl, mask=msk)
                plsc.store_scatter(pendt_v, [pos], t16, mask=msk)
                cnew = cnew + cs[L - 1]

            @pl.when(k + 2 < CPT)
            def _():
                fire_idx(k + 2, si)

            trig = cnew >= CHUNK
            par = nb % 2

            @pl.when(trig & (par == 0))
            def _b0():
                flush_fire(0)

                @pl.when(nb > 0)
                def _d1():
                    drain(1)

            @pl.when(trig & (par == 1))
            def _b1():
                flush_fire(1)
                drain(0)

            cnew = jnp.where(trig, cnew - CHUNK, cnew)
            return cnew, jnp.where(trig, nb + 1, nb)

        def chunk2(m, carry2):
            cnt, nb = carry2
            cnt, nb = do_chunk(2 * m, 0, cnt, nb)
            cnt, nb = do_chunk(2 * m + 1, 1, cnt, nb)
            return cnt, nb

        cnt, nb = lax.fori_loop(0, CPT // 2, chunk2,
                                (jnp.int32(0), jnp.int32(0)))

        trash_s = jnp.zeros((L,), jnp.int32) + soff
        trash_r = jnp.full((L,), TRASH, jnp.int32)
        trash_t = jnp.zeros((L,), jnp.int32)
        for i in range(CHUNK // L):
            base = pl.ds(cnt + i * L, L)
            pends_v[base] = trash_s
            pendr_v[base] = trash_r
            pendt_v[base] = trash_t

        @pl.when(nb % 2 == 0)
        def _t0():
            flush_fire(0)

            @pl.when(nb > 0)
            def _td1():
                drain(1)
            drain(0)

        @pl.when(nb % 2 == 1)
        def _t1():
            flush_fire(1)
            drain(0)
            drain(1)

        plsc.subcore_barrier()
        pltpu.sync_copy(acc_sh.at[arows], acc_out.at[c, p, arows])
        plsc.subcore_barrier()
        return carry

    lax.fori_loop(0, 2, phase, 0)


_edge_kernel = functools.partial(
    pl.kernel,
    out_type=jax.ShapeDtypeStruct((2, 2, NPH, D), jnp.float32),
    mesh=plsc.VectorSubcoreMesh(
        core_axis_name="c", subcore_axis_name="s", num_cores=2,
        num_subcores=NS),
    compiler_params=pltpu.CompilerParams(needs_layout_passes=False),
    scratch_types=[
        pltpu.VMEM((3, CHUNK), jnp.int32),
        pltpu.VMEM((3, CHUNK), jnp.int32),
        pltpu.VMEM((TP, H), jnp.float32),
        pltpu.VMEM((CHUNK, D), jnp.float32),
        pltpu.VMEM((CHUNK, D), jnp.float32),
        pltpu.VMEM((CHUNK,), jnp.int32),
        pltpu.VMEM((CHUNK,), jnp.int32),
        pltpu.VMEM((1, CHUNK), jnp.int32),
        pltpu.VMEM((1, CHUNK), jnp.int32),
        pltpu.VMEM((CHUNK,), jnp.int32),
        pltpu.VMEM((CHUNK,), jnp.int32),
        pltpu.VMEM((2 * CHUNK,), jnp.int32),
        pltpu.VMEM((2 * CHUNK,), jnp.int32),
        pltpu.VMEM((2 * CHUNK,), jnp.int32),
        pltpu.VMEM_SHARED((NPH, D), jnp.float32),
        pltpu.SemaphoreType.DMA,
        pltpu.SemaphoreType.DMA,
        pltpu.SemaphoreType.DMA,
        pltpu.SemaphoreType.DMA,
    ],
)(_edge_body)


def _div_body(acc_ref, out_ref):
    for h in range(2):
      for p in range(2):
        n = acc_ref[h, p, :RH, :H]
        d = acc_ref[h, p, :RH, H:]
        safe = jnp.where(d > 0.0, d, 1.0)
        out_ref[p * RH:(p + 1) * RH, h * H:(h + 1) * H] = jnp.where(
            d > 0.0, n / safe, 0.0)


def _divide(acc):
    return pl.pallas_call(
        _div_body,
        out_shape=jax.ShapeDtypeStruct((N, D), jnp.float32),
    )(acc)


def kernel(x, sender_idx, receiver_idx, type_idx, V_proj_sender,
           V_gate_sender, E_gate, B_message, B_gate_pre):
    egs = E_gate.reshape(T, 2, H).transpose(1, 0, 2)
    egt0 = jnp.zeros((2, TP, H), jnp.float32).at[:, :T, :].set(egs)
    cmb, egt = _project(x, V_proj_sender, V_gate_sender, B_message,
                        B_gate_pre, egt0)
    cmb = cmb.reshape(2 * N, D)
    zer = jnp.zeros((NPH, D), jnp.float32)
    pad = EP - E
    sidx = jnp.concatenate(
        [sender_idx, jnp.zeros((pad,), jnp.int32)]).reshape(NCH, CHUNK)
    ridx = jnp.concatenate(
        [receiver_idx, jnp.full((pad,), N, jnp.int32)]).reshape(NCH, CHUNK)
    tidx = jnp.concatenate(
        [type_idx, jnp.zeros((pad,), jnp.int32)]).reshape(NCH, CHUNK)
    idx = jnp.stack([sidx, ridx, tidx], axis=1)
    acc = _edge_kernel(cmb, egt, idx, zer)
    return _divide(acc)
